# baseline (speedup 1.0000x reference)
import functools

import jax
import jax.numpy as jnp
from jax import lax
from jax.experimental import pallas as pl
from jax.experimental.pallas import tpu as pltpu
from jax.experimental.pallas import tpu_sc as plsc

EMBED = 64
CHUNK = 128
LANES = 16
NUM_CORES = 2
NUM_SUBCORES = 16
NUM_WORKERS = NUM_CORES * NUM_SUBCORES


@functools.lru_cache(maxsize=None)
def _make_sc_kernel(n_rows: int, chunks_per_worker: int):
    G = chunks_per_worker
    rows_per_worker = G * CHUNK
    mesh = plsc.VectorSubcoreMesh(core_axis_name="c", subcore_axis_name="s")

    @functools.partial(
        pl.kernel,
        mesh=mesh,
        compiler_params=pltpu.CompilerParams(needs_layout_passes=False,
                                             use_tc_tiling_on_sc=False),
        out_type=jax.ShapeDtypeStruct((n_rows,), jnp.float32),
        scratch_types=[
            pltpu.VMEM((rows_per_worker,), jnp.int32),
            pltpu.VMEM((rows_per_worker,), jnp.int32),
            pltpu.VMEM((rows_per_worker,), jnp.int32),
            pltpu.VMEM((CHUNK, EMBED), jnp.float32),
            pltpu.VMEM((CHUNK, EMBED), jnp.float32),
            pltpu.VMEM((CHUNK, EMBED), jnp.float32),
            pltpu.VMEM((CHUNK, EMBED), jnp.float32),
            pltpu.VMEM((CHUNK, EMBED), jnp.float32),
            pltpu.VMEM((CHUNK, EMBED), jnp.float32),
            pltpu.VMEM((CHUNK,), jnp.float32),
            pltpu.VMEM((CHUNK,), jnp.float32),
            pltpu.SemaphoreType.DMA,
            pltpu.SemaphoreType.DMA,
            pltpu.SemaphoreType.DMA,
            pltpu.SemaphoreType.DMA,
        ],
    )
    def sc_kernel(table, idxl, idxr, idxd, out,
                  idxl_v, idxr_v, idxd_v,
                  l0, l1, r0, r1, d0, d1, s0, s1,
                  gsem0, gsem1, ssem0, ssem1):
        wid = lax.axis_index("s") * NUM_CORES + lax.axis_index("c")
        base = pl.multiple_of(wid * rows_per_worker, CHUNK)

        pltpu.sync_copy(idxl.at[pl.ds(base, rows_per_worker)], idxl_v)
        pltpu.sync_copy(idxr.at[pl.ds(base, rows_per_worker)], idxr_v)
        pltpu.sync_copy(idxd.at[pl.ds(base, rows_per_worker)], idxd_v)

        slots = ((l0, r0, d0, s0, gsem0, ssem0),
                 (l1, r1, d1, s1, gsem1, ssem1))

        def chunk_idx(ref, g):
            return ref.at[pl.ds(pl.multiple_of(g * CHUNK, CHUNK), CHUNK)]

        def fire(g, slot):
            lb, rb, db, _, gsem, _ = slots[slot]
            pltpu.async_copy(table.at[chunk_idx(idxl_v, g)], lb, gsem)
            pltpu.async_copy(table.at[chunk_idx(idxr_v, g)], rb, gsem)
            pltpu.async_copy(table.at[chunk_idx(idxd_v, g)], db, gsem)

        def wait_gather(g, slot):
            lb, rb, db, _, gsem, _ = slots[slot]
            pltpu.make_async_copy(table.at[chunk_idx(idxl_v, g)], lb, gsem).wait()
            pltpu.make_async_copy(table.at[chunk_idx(idxr_v, g)], rb, gsem).wait()
            pltpu.make_async_copy(table.at[chunk_idx(idxd_v, g)], db, gsem).wait()

        def out_slice(g):
            off = pl.multiple_of(base + g * CHUNK, CHUNK)
            return out.at[pl.ds(off, CHUNK)]

        fire(0, 0)
        fire(1, 1)

        lane_iota = lax.iota(jnp.int32, LANES)

        def compute(g, slot):
            lb, rb, db, sb, _, ssem = slots[slot]

            def group_body(t, carry):
                rowbase = t * LANES
                acc = jnp.zeros((LANES,), jnp.float32)
                for rr in range(LANES):
                    row = rowbase + rr
                    part = None
                    for c in range(0, EMBED, LANES):
                        lv = lb[row, pl.ds(c, LANES)]
                        rv = rb[row, pl.ds(c, LANES)]
                        dv = db[row, pl.ds(c, LANES)]
                        diff = jnp.minimum(lv, rv) - dv
                        sq = diff * diff
                        part = sq if part is None else part + sq
                    s = jnp.sum(part)
                    acc = acc + jnp.where(lane_iota == rr, s, 0.0)
                x = acc + 1e-12
                bits = plsc.bitcast(x, jnp.int32)
                y = plsc.bitcast(jnp.int32(0x5F3759DF) - (bits >> 1),
                                 jnp.float32)
                for _ in range(3):
                    y = y * (1.5 - 0.5 * x * y * y)
                sb[pl.ds(pl.multiple_of(t * LANES, LANES), LANES)] = -(x * y)
                return carry

            lax.fori_loop(0, CHUNK // LANES, group_body, None)
            pltpu.async_copy(sb, out_slice(g), ssem)

        def loop_body(i, carry):
            for slot in (0, 1):
                g = i * 2 + slot
                wait_gather(g, slot)

                @pl.when(g >= 2)
                def _drain_score():
                    sb, ssem = slots[slot][3], slots[slot][5]
                    pltpu.make_async_copy(sb, out_slice(g - 2), ssem).wait()

                compute(g, slot)

                @pl.when(g + 2 < G)
                def _fire_next():
                    fire(g + 2, slot)
            return carry

        lax.fori_loop(0, G // 2, loop_body, None)

        for slot in (0, 1):
            g_last = G - 2 + slot
            sb, ssem = slots[slot][3], slots[slot][5]
            pltpu.make_async_copy(sb, out_slice(g_last), ssem).wait()

    return sc_kernel


def kernel(x, table):
    bs, num_axioms, ents = x.shape
    assert ents == 3
    n_rows = bs * num_axioms
    assert n_rows % (NUM_WORKERS * CHUNK) == 0
    chunks_per_worker = n_rows // (NUM_WORKERS * CHUNK)

    flat = x.reshape(n_rows, ents).astype(jnp.int32)
    idxl = flat[:, 0]
    idxr = flat[:, 1]
    idxd = flat[:, 2]

    sc = _make_sc_kernel(n_rows, chunks_per_worker)
    scores = sc(table.astype(jnp.float32), idxl, idxr, idxd)
    return scores.reshape(bs, num_axioms)
